# baseline (speedup 1.0000x reference)
import jax
import jax.numpy as jnp
from jax import lax
from jax.experimental import pallas as pl
from jax.experimental.pallas import tpu as pltpu
from jax.experimental.pallas import tpu_sc as plsc

F32 = jnp.float32

NUM_NODES = 500
BATCH = 100
N = NUM_NODES * BATCH
E = 600000
NODE_DIM = 64
EMB = 128
EDGE_DIM = 16
NLAYER = 3

QW = 32
NQ = EMB // QW
EB = 128
NTILE = 32
EPAD = ((E + EB * NTILE - 1) // (EB * NTILE)) * (EB * NTILE)
NB = EPAD // EB
BPT = NB // NTILE
ACCR = ((N + 128) // 128) * 128
RPT = ACCR // 16
ZR = 68
ZFULL = RPT // ZR
ZREM = RPT - ZFULL * ZR


def _leaky(v):
    return jnp.where(v >= 0, v, 0.1 * v)


def _encoder_body(x_ref, w_ref, cb_ref, lw_ref, lb_ref,
                  h_ref, q0_ref, q1_ref, q2_ref, q3_ref):
    xb = x_ref[...]
    nn = xb.shape[0] // 2

    def shifts(m):
        z = jnp.zeros((nn, 1), F32)
        left = jnp.concatenate([z, m[:, :-1]], axis=1)
        right = jnp.concatenate([m[:, 1:], z], axis=1)
        return jnp.concatenate([left, m, right], axis=0)

    xcat = jnp.concatenate([shifts(xb[:nn]), shifts(xb[nn:])], axis=1)
    o = jnp.dot(w_ref[...], xcat, preferred_element_type=F32)
    o = _leaky(o + cb_ref[...])
    lw = lw_ref[...]
    lb = lb_ref[...]
    h0 = _leaky(jnp.dot(o[:, :NODE_DIM], lw, preferred_element_type=F32) + lb)
    h1 = _leaky(jnp.dot(o[:, NODE_DIM:], lw, preferred_element_type=F32) + lb)
    h = jnp.concatenate([h0, h1], axis=0)
    h_ref[...] = h
    q0_ref[...] = h[:, 0 * QW:1 * QW]
    q1_ref[...] = h[:, 1 * QW:2 * QW]
    q2_ref[...] = h[:, 2 * QW:3 * QW]
    q3_ref[...] = h[:, 3 * QW:4 * QW]


def _encoder(x, wcat, cb, lw, lb):
    nrow = 2 * NUM_NODES
    grid = (BATCH // 2,)
    qspec = pl.BlockSpec((nrow, QW), lambda i: (i, 0))
    return pl.pallas_call(
        _encoder_body,
        grid=grid,
        in_specs=[
            pl.BlockSpec((nrow, NODE_DIM), lambda i: (i, 0)),
            pl.BlockSpec((NUM_NODES, 3 * NUM_NODES), lambda i: (0, 0)),
            pl.BlockSpec((NUM_NODES, 1), lambda i: (0, 0)),
            pl.BlockSpec((NODE_DIM, EMB), lambda i: (0, 0)),
            pl.BlockSpec((1, EMB), lambda i: (0, 0)),
        ],
        out_specs=[pl.BlockSpec((nrow, EMB), lambda i: (i, 0)),
                   qspec, qspec, qspec, qspec],
        out_shape=[jax.ShapeDtypeStruct((N, EMB), F32)] +
                  [jax.ShapeDtypeStruct((N, QW), F32)] * NQ,
    )(x, wcat, cb, lw, lb)


def _ee_body(a4_ref, w0_ref, w1_ref, w2_ref, w3_ref,
             b0_ref, b1_ref, b2_ref, b3_ref,
             q0_ref, q1_ref, q2_ref, q3_ref):
    a4 = a4_ref[...]
    for wr, br, qr in ((w0_ref, b0_ref, q0_ref), (w1_ref, b1_ref, q1_ref),
                       (w2_ref, b2_ref, q2_ref), (w3_ref, b3_ref, q3_ref)):
        v = jnp.dot(a4, wr[...], preferred_element_type=F32)
        qr[...] = jnp.maximum(v + br[...], 0.0)


def _edge_embed(a4, w4s, b4s):
    blk = 1536
    grid = (EPAD // 4 // blk,)
    qspec = pl.BlockSpec((blk, EMB), lambda i: (i, 0))
    wspec = pl.BlockSpec((4 * EDGE_DIM, EMB), lambda i: (0, 0))
    bspec = pl.BlockSpec((1, EMB), lambda i: (0, 0))
    return pl.pallas_call(
        _ee_body,
        grid=grid,
        in_specs=[pl.BlockSpec((blk, 4 * EDGE_DIM), lambda i: (i, 0)),
                  wspec, wspec, wspec, wspec,
                  bspec, bspec, bspec, bspec],
        out_specs=[qspec] * NQ,
        out_shape=[jax.ShapeDtypeStruct((EPAD // 4, EMB), F32)] * NQ,
    )(a4, *w4s, *b4s)


def _sc_body(idx_ref,
             h0_ref, h1_ref, h2_ref, h3_ref,
             e0_ref, e1_ref, e2_ref, e3_ref,
             apk_ref,
             iv, hbuf, ebuf, mbuf, zbuf, acc, semi, semg, seme, sems):
    c = lax.axis_index("c")
    s = lax.axis_index("s")
    wid = s * 2 + c
    b0 = wid * BPT

    def zfill(i, _):
        zbuf[i, pl.ds(0, 16)] = jnp.zeros((16,), F32)
        zbuf[i, pl.ds(16, 16)] = jnp.zeros((16,), F32)
        return 0
    lax.fori_loop(0, ZR, zfill, 0)

    hq = (h0_ref, h1_ref, h2_ref, h3_ref)
    eq = (e0_ref, e1_ref, e2_ref, e3_ref)

    for q in range(NQ):
        base = s * RPT
        for j in range(ZFULL):
            pltpu.sync_copy(zbuf, acc.at[pl.ds(base + j * ZR, ZR)])
        if ZREM:
            pltpu.sync_copy(zbuf.at[pl.ds(0, ZREM)],
                            acc.at[pl.ds(base + ZFULL * ZR, ZREM)])
        plsc.subcore_barrier()

        def idx_desc(j):
            sl = lax.rem(j, 4)
            return pltpu.make_async_copy(
                idx_ref.at[pl.ds(b0 + j, 1)], iv.at[pl.ds(sl, 1)],
                semi.at[sl])

        def gather_desc(j):
            return pltpu.make_async_copy(
                hq[q].at[iv.at[lax.rem(j, 4), 0]],
                hbuf.at[lax.rem(j, 2)], semg.at[lax.rem(j, 2)])

        def ee_desc(j):
            return pltpu.make_async_copy(
                eq[q].at[pl.ds((b0 + j) * QW, QW)],
                ebuf.at[lax.rem(j, 2)], seme.at[lax.rem(j, 2)])

        def scat_desc(j):
            return pltpu.make_async_copy(
                mbuf.at[lax.rem(j, 2)], acc.at[iv.at[lax.rem(j, 4), 1]],
                sems.at[lax.rem(j, 2)])

        idx_desc(0).start()
        idx_desc(1).start()
        idx_desc(0).wait()
        gather_desc(0).start()
        ee_desc(0).start()

        def batch_body(i, _):
            sl = lax.rem(i, 2)

            @pl.when(i >= 2)
            def _():
                scat_desc(i - 2).wait()

            @pl.when(i + 2 < BPT)
            def _():
                idx_desc(i + 2).start()

            @pl.when(i + 1 < BPT)
            def _():
                idx_desc(i + 1).wait()
                gather_desc(i + 1).start()
                ee_desc(i + 1).start()

            gather_desc(i).wait()
            ee_desc(i).wait()

            def ebody(t, _):
                for dr in range(4):
                    er = t * 4 + dr
                    for j in range(4):
                        for kk in (0, 16):
                            v = (hbuf[sl, er * 4 + j, pl.ds(kk, 16)] +
                                 ebuf[sl, er, pl.ds(j * QW + kk, 16)])
                            mbuf[sl, er * 4 + j, pl.ds(kk, 16)] = (
                                jnp.maximum(v, 0.0))
                return 0
            lax.fori_loop(0, QW // 4, ebody, 0)

            pltpu.async_copy(mbuf.at[sl], acc.at[iv.at[lax.rem(i, 4), 1]],
                             sems.at[lax.rem(i, 2)], add=True)
            return 0
        lax.fori_loop(0, BPT, batch_body, 0)

        scat_desc(BPT - 2).wait()
        scat_desc(BPT - 1).wait()

        plsc.subcore_barrier()
        pltpu.sync_copy(acc.at[pl.ds(s * RPT, RPT)],
                        apk_ref.at[c, pl.ds(s * RPT, RPT),
                                   pl.ds(q * QW, QW)])
        plsc.subcore_barrier()


def _sc_messages(idx2, hqs, eqs):
    mesh = plsc.VectorSubcoreMesh(core_axis_name="c", subcore_axis_name="s")
    fn = pl.kernel(
        _sc_body,
        out_type=jax.ShapeDtypeStruct((2, ACCR, EMB), F32),
        mesh=mesh,
        compiler_params=pltpu.CompilerParams(use_tc_tiling_on_sc=False),
        scratch_types=[
            pltpu.VMEM((4, 2, EB), jnp.int32),
            pltpu.VMEM((2, EB, QW), F32),
            pltpu.VMEM((2, QW, EMB), F32),
            pltpu.VMEM((2, EB, QW), F32),
            pltpu.VMEM((ZR, QW), F32),
            pltpu.VMEM_SHARED((ACCR, QW), F32),
            pltpu.SemaphoreType.DMA((4,)),
            pltpu.SemaphoreType.DMA((2,)),
            pltpu.SemaphoreType.DMA((2,)),
            pltpu.SemaphoreType.DMA((2,)),
        ],
    )
    return fn(idx2, *hqs, *eqs)


def _mlpa_body(h_ref, a_ref, k_ref,
               w1_ref, b1_ref, z_ref, s_ref, s2_ref):
    agg = a_ref[0] + a_ref[1]
    t = h_ref[...] * k_ref[...] + agg
    z = jnp.dot(t, w1_ref[...], preferred_element_type=F32) + b1_ref[...]
    z_ref[...] = z
    s = jnp.sum(z, axis=0, keepdims=True)
    s2 = jnp.sum(z * z, axis=0, keepdims=True)

    @pl.when(pl.program_id(0) == 0)
    def _():
        s_ref[...] = s
        s2_ref[...] = s2

    @pl.when(pl.program_id(0) != 0)
    def _():
        s_ref[...] += s
        s2_ref[...] += s2


def _mlp_a(h, agg, kmul, w1, b1):
    rblk = 2000
    grid = (N // rblk,)
    return pl.pallas_call(
        _mlpa_body,
        grid=grid,
        in_specs=[
            pl.BlockSpec((rblk, EMB), lambda i: (i, 0)),
            pl.BlockSpec((2, rblk, EMB), lambda i: (0, i, 0)),
            pl.BlockSpec((1, 1), lambda i: (0, 0)),
            pl.BlockSpec((EMB, 2 * EMB), lambda i: (0, 0)),
            pl.BlockSpec((1, 2 * EMB), lambda i: (0, 0)),
        ],
        out_specs=[
            pl.BlockSpec((rblk, 2 * EMB), lambda i: (i, 0)),
            pl.BlockSpec((1, 2 * EMB), lambda i: (0, 0)),
            pl.BlockSpec((1, 2 * EMB), lambda i: (0, 0)),
        ],
        out_shape=[
            jax.ShapeDtypeStruct((N, 2 * EMB), F32),
            jax.ShapeDtypeStruct((1, 2 * EMB), F32),
            jax.ShapeDtypeStruct((1, 2 * EMB), F32),
        ],
    )(h, agg, kmul, w1, b1)


def _mlpb_body(z_ref, h_ref, jk_ref, s_ref, s2_ref, bg_ref, bb_ref,
               w2_ref, b2_ref, lg_ref, lb_ref,
               hn_ref, jko_ref, q0_ref, q1_ref, q2_ref, q3_ref):
    inv_n = 1.0 / N
    mu = s_ref[...] * inv_n
    var = s2_ref[...] * inv_n - mu * mu
    scale = bg_ref[...] * lax.rsqrt(var + 1e-5)
    zb = (z_ref[...] - mu) * scale + bb_ref[...]
    zb = jnp.maximum(zb, 0.0)
    y = jnp.dot(zb, w2_ref[...], preferred_element_type=F32) + b2_ref[...]
    m = jnp.mean(y, axis=1, keepdims=True)
    v = jnp.mean(y * y, axis=1, keepdims=True) - m * m
    y = (y - m) * lax.rsqrt(v + 1e-5) * lg_ref[...] + lb_ref[...]
    y = _leaky(y)
    hn = y + h_ref[...]
    hn_ref[...] = hn
    jko_ref[...] = jk_ref[...] + hn
    q0_ref[...] = hn[:, 0 * QW:1 * QW]
    q1_ref[...] = hn[:, 1 * QW:2 * QW]
    q2_ref[...] = hn[:, 2 * QW:3 * QW]
    q3_ref[...] = hn[:, 3 * QW:4 * QW]


def _mlp_b(z, h, jk, s, s2, bg, bb, w2, b2, lg, lb):
    rblk = 2000
    grid = (N // rblk,)
    qspec = pl.BlockSpec((rblk, QW), lambda i: (i, 0))
    row128 = pl.BlockSpec((1, EMB), lambda i: (0, 0))
    row256 = pl.BlockSpec((1, 2 * EMB), lambda i: (0, 0))
    return pl.pallas_call(
        _mlpb_body,
        grid=grid,
        in_specs=[
            pl.BlockSpec((rblk, 2 * EMB), lambda i: (i, 0)),
            pl.BlockSpec((rblk, EMB), lambda i: (i, 0)),
            pl.BlockSpec((rblk, EMB), lambda i: (i, 0)),
            row256, row256, row256, row256,
            pl.BlockSpec((2 * EMB, EMB), lambda i: (0, 0)),
            row128, row128, row128,
        ],
        out_specs=[pl.BlockSpec((rblk, EMB), lambda i: (i, 0)),
                   pl.BlockSpec((rblk, EMB), lambda i: (i, 0)),
                   qspec, qspec, qspec, qspec],
        out_shape=[jax.ShapeDtypeStruct((N, EMB), F32),
                   jax.ShapeDtypeStruct((N, EMB), F32)] +
                  [jax.ShapeDtypeStruct((N, QW), F32)] * NQ,
    )(z, h, jk, s, s2, bg, bb, w2, b2, lg, lb)


@jax.jit
def kernel(x, edge_index, edge_attr, conv_w, conv_b, enc_lin_w, enc_lin_b,
           mlp_w1, mlp_b1, bn_gamma, bn_beta, mlp_w2, mlp_b2, eps,
           edge_w, edge_b, ln_gamma, ln_beta):
    wcat = jnp.concatenate([conv_w[:, :, k] for k in range(3)], axis=1)
    cb = conv_b[:, None]
    lb = enc_lin_b[None, :]

    row = edge_index[0]
    col = edge_index[1]
    pad = EPAD - E
    rows2d = jnp.concatenate(
        [row, jnp.zeros((pad,), jnp.int32)]).reshape(NB, EB)
    cols2d = jnp.concatenate(
        [col, jnp.full((pad,), N, jnp.int32)]).reshape(NB, EB)
    idx2 = jnp.stack([rows2d, cols2d], axis=1)
    a4 = jnp.concatenate(
        [edge_attr, jnp.zeros((pad, EDGE_DIM), F32)],
        axis=0).reshape(EPAD // 4, 4 * EDGE_DIM)
    eye4 = jnp.eye(4, dtype=F32)

    eqs_all = []
    for l in range(NLAYER):
        w4s = [jnp.kron(eye4, edge_w[l][:, q * QW:(q + 1) * QW])
               for q in range(NQ)]
        b4s = [jnp.tile(edge_b[l][q * QW:(q + 1) * QW], 4)[None, :]
               for q in range(NQ)]
        eqs_all.append(_edge_embed(a4, w4s, b4s))

    h, *hqs = _encoder(x, wcat, cb, enc_lin_w, lb)
    jk = h
    for l in range(NLAYER):
        agg = _sc_messages(idx2, hqs, eqs_all[l])
        kmul = (1.0 + eps[l]).reshape(1, 1)
        z, s, s2 = _mlp_a(h, agg, kmul, mlp_w1[l], mlp_b1[l][None, :])
        h, jk, *hqs = _mlp_b(z, h, jk, s, s2,
                             bn_gamma[l][None, :], bn_beta[l][None, :],
                             mlp_w2[l], mlp_b2[l][None, :],
                             ln_gamma[l][None, :], ln_beta[l][None, :])
    return jk


# speedup vs baseline: 1.0069x; 1.0069x over previous
import jax
import jax.numpy as jnp
from jax import lax
from jax.experimental import pallas as pl
from jax.experimental.pallas import tpu as pltpu
from jax.experimental.pallas import tpu_sc as plsc

F32 = jnp.float32

NUM_NODES = 500
BATCH = 100
N = NUM_NODES * BATCH
E = 600000
NODE_DIM = 64
EMB = 128
EDGE_DIM = 16
NLAYER = 3

QW = 32
NQ = EMB // QW
EB = 128
NTILE = 32
EPAD = ((E + EB * NTILE - 1) // (EB * NTILE)) * (EB * NTILE)
NB = EPAD // EB
BPT = NB // NTILE
ACCR = ((N + 128) // 128) * 128
RPT = ACCR // 16
ZR = 136
ZFULL = RPT // ZR
ZREM = RPT - ZFULL * ZR


def _leaky(v):
    return jnp.where(v >= 0, v, 0.1 * v)


def _encoder_body(x_ref, w_ref, cb_ref, lw_ref, lb_ref,
                  h_ref, q0_ref, q1_ref, q2_ref, q3_ref):
    xb = x_ref[...]
    nn = xb.shape[0] // 2

    def shifts(m):
        z = jnp.zeros((nn, 1), F32)
        left = jnp.concatenate([z, m[:, :-1]], axis=1)
        right = jnp.concatenate([m[:, 1:], z], axis=1)
        return jnp.concatenate([left, m, right], axis=0)

    xcat = jnp.concatenate([shifts(xb[:nn]), shifts(xb[nn:])], axis=1)
    o = jnp.dot(w_ref[...], xcat, preferred_element_type=F32)
    o = _leaky(o + cb_ref[...])
    lw = lw_ref[...]
    lb = lb_ref[...]
    h0 = _leaky(jnp.dot(o[:, :NODE_DIM], lw, preferred_element_type=F32) + lb)
    h1 = _leaky(jnp.dot(o[:, NODE_DIM:], lw, preferred_element_type=F32) + lb)
    h = jnp.concatenate([h0, h1], axis=0)
    h_ref[...] = h
    q0_ref[...] = h[:, 0 * QW:1 * QW]
    q1_ref[...] = h[:, 1 * QW:2 * QW]
    q2_ref[...] = h[:, 2 * QW:3 * QW]
    q3_ref[...] = h[:, 3 * QW:4 * QW]


def _encoder(x, wcat, cb, lw, lb):
    nrow = 2 * NUM_NODES
    grid = (BATCH // 2,)
    qspec = pl.BlockSpec((nrow, QW), lambda i: (i, 0))
    return pl.pallas_call(
        _encoder_body,
        grid=grid,
        in_specs=[
            pl.BlockSpec((nrow, NODE_DIM), lambda i: (i, 0)),
            pl.BlockSpec((NUM_NODES, 3 * NUM_NODES), lambda i: (0, 0)),
            pl.BlockSpec((NUM_NODES, 1), lambda i: (0, 0)),
            pl.BlockSpec((NODE_DIM, EMB), lambda i: (0, 0)),
            pl.BlockSpec((1, EMB), lambda i: (0, 0)),
        ],
        out_specs=[pl.BlockSpec((nrow, EMB), lambda i: (i, 0)),
                   qspec, qspec, qspec, qspec],
        out_shape=[jax.ShapeDtypeStruct((N, EMB), F32)] +
                  [jax.ShapeDtypeStruct((N, QW), F32)] * NQ,
    )(x, wcat, cb, lw, lb)


def _ee_body(a4_ref, w0_ref, w1_ref, w2_ref, w3_ref,
             b0_ref, b1_ref, b2_ref, b3_ref,
             q0_ref, q1_ref, q2_ref, q3_ref):
    a4 = a4_ref[...]
    for wr, br, qr in ((w0_ref, b0_ref, q0_ref), (w1_ref, b1_ref, q1_ref),
                       (w2_ref, b2_ref, q2_ref), (w3_ref, b3_ref, q3_ref)):
        v = jnp.dot(a4, wr[...], preferred_element_type=F32)
        qr[...] = jnp.maximum(v + br[...], 0.0)


def _edge_embed(a4, w4s, b4s):
    blk = 1536
    grid = (EPAD // 4 // blk,)
    qspec = pl.BlockSpec((blk, EMB), lambda i: (i, 0))
    wspec = pl.BlockSpec((4 * EDGE_DIM, EMB), lambda i: (0, 0))
    bspec = pl.BlockSpec((1, EMB), lambda i: (0, 0))
    return pl.pallas_call(
        _ee_body,
        grid=grid,
        in_specs=[pl.BlockSpec((blk, 4 * EDGE_DIM), lambda i: (i, 0)),
                  wspec, wspec, wspec, wspec,
                  bspec, bspec, bspec, bspec],
        out_specs=[qspec] * NQ,
        out_shape=[jax.ShapeDtypeStruct((EPAD // 4, EMB), F32)] * NQ,
    )(a4, *w4s, *b4s)


def _sc_body(idx_ref,
             h0_ref, h1_ref, h2_ref, h3_ref,
             e0_ref, e1_ref, e2_ref, e3_ref,
             apk_ref,
             iv, hbuf, ebuf, mbuf, zbuf, acc, semi, semg, seme, sems):
    c = lax.axis_index("c")
    s = lax.axis_index("s")
    wid = s * 2 + c
    b0 = wid * BPT

    def zfill(i, _):
        zbuf[i, pl.ds(0, 16)] = jnp.zeros((16,), F32)
        zbuf[i, pl.ds(16, 16)] = jnp.zeros((16,), F32)
        return 0
    lax.fori_loop(0, ZR, zfill, 0)

    hq = (h0_ref, h1_ref, h2_ref, h3_ref)
    eq = (e0_ref, e1_ref, e2_ref, e3_ref)

    for q in range(NQ):
        base = s * RPT
        for j in range(ZFULL):
            pltpu.async_copy(zbuf, acc.at[pl.ds(base + j * ZR, ZR)],
                             semg.at[0])
        for j in range(ZFULL):
            pltpu.make_async_copy(zbuf, acc.at[pl.ds(base + j * ZR, ZR)],
                                  semg.at[0]).wait()
        if ZREM:
            pltpu.sync_copy(zbuf.at[pl.ds(0, ZREM)],
                            acc.at[pl.ds(base + ZFULL * ZR, ZREM)])
        plsc.subcore_barrier()

        def idx_desc(j):
            sl = lax.rem(j, 4)
            return pltpu.make_async_copy(
                idx_ref.at[pl.ds(b0 + j, 1)], iv.at[pl.ds(sl, 1)],
                semi.at[sl])

        def gather_desc(j):
            return pltpu.make_async_copy(
                hq[q].at[iv.at[lax.rem(j, 4), 0]],
                hbuf.at[lax.rem(j, 2)], semg.at[lax.rem(j, 2)])

        def ee_desc(j):
            return pltpu.make_async_copy(
                eq[q].at[pl.ds((b0 + j) * QW, QW)],
                ebuf.at[lax.rem(j, 2)], seme.at[lax.rem(j, 2)])

        def scat_desc(j):
            return pltpu.make_async_copy(
                mbuf.at[lax.rem(j, 2)], acc.at[iv.at[lax.rem(j, 4), 1]],
                sems.at[lax.rem(j, 2)])

        idx_desc(0).start()
        idx_desc(1).start()
        idx_desc(0).wait()
        gather_desc(0).start()
        ee_desc(0).start()

        def batch_body(i, _):
            sl = lax.rem(i, 2)

            @pl.when(i >= 2)
            def _():
                scat_desc(i - 2).wait()

            @pl.when(i + 2 < BPT)
            def _():
                idx_desc(i + 2).start()

            @pl.when(i + 1 < BPT)
            def _():
                idx_desc(i + 1).wait()
                gather_desc(i + 1).start()
                ee_desc(i + 1).start()

            gather_desc(i).wait()
            ee_desc(i).wait()

            def ebody(t, _):
                for dr in range(4):
                    er = t * 4 + dr
                    for j in range(4):
                        for kk in (0, 16):
                            v = (hbuf[sl, er * 4 + j, pl.ds(kk, 16)] +
                                 ebuf[sl, er, pl.ds(j * QW + kk, 16)])
                            mbuf[sl, er * 4 + j, pl.ds(kk, 16)] = (
                                jnp.maximum(v, 0.0))
                return 0
            lax.fori_loop(0, QW // 4, ebody, 0)

            pltpu.async_copy(mbuf.at[sl], acc.at[iv.at[lax.rem(i, 4), 1]],
                             sems.at[lax.rem(i, 2)], add=True)
            return 0
        lax.fori_loop(0, BPT, batch_body, 0)

        scat_desc(BPT - 2).wait()
        scat_desc(BPT - 1).wait()

        plsc.subcore_barrier()
        pltpu.sync_copy(acc.at[pl.ds(s * RPT, RPT)],
                        apk_ref.at[c, pl.ds(s * RPT, RPT),
                                   pl.ds(q * QW, QW)])
        plsc.subcore_barrier()


def _sc_messages(idx2, hqs, eqs):
    mesh = plsc.VectorSubcoreMesh(core_axis_name="c", subcore_axis_name="s")
    fn = pl.kernel(
        _sc_body,
        out_type=jax.ShapeDtypeStruct((2, ACCR, EMB), F32),
        mesh=mesh,
        compiler_params=pltpu.CompilerParams(use_tc_tiling_on_sc=False),
        scratch_types=[
            pltpu.VMEM((4, 2, EB), jnp.int32),
            pltpu.VMEM((2, EB, QW), F32),
            pltpu.VMEM((2, QW, EMB), F32),
            pltpu.VMEM((2, EB, QW), F32),
            pltpu.VMEM((ZR, QW), F32),
            pltpu.VMEM_SHARED((ACCR, QW), F32),
            pltpu.SemaphoreType.DMA((4,)),
            pltpu.SemaphoreType.DMA((2,)),
            pltpu.SemaphoreType.DMA((2,)),
            pltpu.SemaphoreType.DMA((2,)),
        ],
    )
    return fn(idx2, *hqs, *eqs)


def _mlpa_body(h_ref, a_ref, k_ref,
               w1_ref, b1_ref, z_ref, s_ref, s2_ref):
    agg = a_ref[0] + a_ref[1]
    t = h_ref[...] * k_ref[...] + agg
    z = jnp.dot(t, w1_ref[...], preferred_element_type=F32) + b1_ref[...]
    z_ref[...] = z
    s = jnp.sum(z, axis=0, keepdims=True)
    s2 = jnp.sum(z * z, axis=0, keepdims=True)

    @pl.when(pl.program_id(0) == 0)
    def _():
        s_ref[...] = s
        s2_ref[...] = s2

    @pl.when(pl.program_id(0) != 0)
    def _():
        s_ref[...] += s
        s2_ref[...] += s2


def _mlp_a(h, agg, kmul, w1, b1):
    rblk = 2000
    grid = (N // rblk,)
    return pl.pallas_call(
        _mlpa_body,
        grid=grid,
        in_specs=[
            pl.BlockSpec((rblk, EMB), lambda i: (i, 0)),
            pl.BlockSpec((2, rblk, EMB), lambda i: (0, i, 0)),
            pl.BlockSpec((1, 1), lambda i: (0, 0)),
            pl.BlockSpec((EMB, 2 * EMB), lambda i: (0, 0)),
            pl.BlockSpec((1, 2 * EMB), lambda i: (0, 0)),
        ],
        out_specs=[
            pl.BlockSpec((rblk, 2 * EMB), lambda i: (i, 0)),
            pl.BlockSpec((1, 2 * EMB), lambda i: (0, 0)),
            pl.BlockSpec((1, 2 * EMB), lambda i: (0, 0)),
        ],
        out_shape=[
            jax.ShapeDtypeStruct((N, 2 * EMB), F32),
            jax.ShapeDtypeStruct((1, 2 * EMB), F32),
            jax.ShapeDtypeStruct((1, 2 * EMB), F32),
        ],
    )(h, agg, kmul, w1, b1)


def _mlpb_body(z_ref, h_ref, jk_ref, s_ref, s2_ref, bg_ref, bb_ref,
               w2_ref, b2_ref, lg_ref, lb_ref,
               hn_ref, jko_ref, q0_ref, q1_ref, q2_ref, q3_ref):
    inv_n = 1.0 / N
    mu = s_ref[...] * inv_n
    var = s2_ref[...] * inv_n - mu * mu
    scale = bg_ref[...] * lax.rsqrt(var + 1e-5)
    zb = (z_ref[...] - mu) * scale + bb_ref[...]
    zb = jnp.maximum(zb, 0.0)
    y = jnp.dot(zb, w2_ref[...], preferred_element_type=F32) + b2_ref[...]
    m = jnp.mean(y, axis=1, keepdims=True)
    v = jnp.mean(y * y, axis=1, keepdims=True) - m * m
    y = (y - m) * lax.rsqrt(v + 1e-5) * lg_ref[...] + lb_ref[...]
    y = _leaky(y)
    hn = y + h_ref[...]
    hn_ref[...] = hn
    jko_ref[...] = jk_ref[...] + hn
    q0_ref[...] = hn[:, 0 * QW:1 * QW]
    q1_ref[...] = hn[:, 1 * QW:2 * QW]
    q2_ref[...] = hn[:, 2 * QW:3 * QW]
    q3_ref[...] = hn[:, 3 * QW:4 * QW]


def _mlp_b(z, h, jk, s, s2, bg, bb, w2, b2, lg, lb):
    rblk = 2000
    grid = (N // rblk,)
    qspec = pl.BlockSpec((rblk, QW), lambda i: (i, 0))
    row128 = pl.BlockSpec((1, EMB), lambda i: (0, 0))
    row256 = pl.BlockSpec((1, 2 * EMB), lambda i: (0, 0))
    return pl.pallas_call(
        _mlpb_body,
        grid=grid,
        in_specs=[
            pl.BlockSpec((rblk, 2 * EMB), lambda i: (i, 0)),
            pl.BlockSpec((rblk, EMB), lambda i: (i, 0)),
            pl.BlockSpec((rblk, EMB), lambda i: (i, 0)),
            row256, row256, row256, row256,
            pl.BlockSpec((2 * EMB, EMB), lambda i: (0, 0)),
            row128, row128, row128,
        ],
        out_specs=[pl.BlockSpec((rblk, EMB), lambda i: (i, 0)),
                   pl.BlockSpec((rblk, EMB), lambda i: (i, 0)),
                   qspec, qspec, qspec, qspec],
        out_shape=[jax.ShapeDtypeStruct((N, EMB), F32),
                   jax.ShapeDtypeStruct((N, EMB), F32)] +
                  [jax.ShapeDtypeStruct((N, QW), F32)] * NQ,
    )(z, h, jk, s, s2, bg, bb, w2, b2, lg, lb)


@jax.jit
def kernel(x, edge_index, edge_attr, conv_w, conv_b, enc_lin_w, enc_lin_b,
           mlp_w1, mlp_b1, bn_gamma, bn_beta, mlp_w2, mlp_b2, eps,
           edge_w, edge_b, ln_gamma, ln_beta):
    wcat = jnp.concatenate([conv_w[:, :, k] for k in range(3)], axis=1)
    cb = conv_b[:, None]
    lb = enc_lin_b[None, :]

    row = edge_index[0]
    col = edge_index[1]
    pad = EPAD - E
    rows2d = jnp.concatenate(
        [row, jnp.zeros((pad,), jnp.int32)]).reshape(NB, EB)
    cols2d = jnp.concatenate(
        [col, jnp.full((pad,), N, jnp.int32)]).reshape(NB, EB)
    idx2 = jnp.stack([rows2d, cols2d], axis=1)
    a4 = jnp.concatenate(
        [edge_attr, jnp.zeros((pad, EDGE_DIM), F32)],
        axis=0).reshape(EPAD // 4, 4 * EDGE_DIM)
    eye4 = jnp.eye(4, dtype=F32)

    eqs_all = []
    for l in range(NLAYER):
        w4s = [jnp.kron(eye4, edge_w[l][:, q * QW:(q + 1) * QW])
               for q in range(NQ)]
        b4s = [jnp.tile(edge_b[l][q * QW:(q + 1) * QW], 4)[None, :]
               for q in range(NQ)]
        eqs_all.append(_edge_embed(a4, w4s, b4s))

    h, *hqs = _encoder(x, wcat, cb, enc_lin_w, lb)
    jk = h
    for l in range(NLAYER):
        agg = _sc_messages(idx2, hqs, eqs_all[l])
        kmul = (1.0 + eps[l]).reshape(1, 1)
        z, s, s2 = _mlp_a(h, agg, kmul, mlp_w1[l], mlp_b1[l][None, :])
        h, jk, *hqs = _mlp_b(z, h, jk, s, s2,
                             bn_gamma[l][None, :], bn_beta[l][None, :],
                             mlp_w2[l], mlp_b2[l][None, :],
                             ln_gamma[l][None, :], ln_beta[l][None, :])
    return jk


# speedup vs baseline: 1.4153x; 1.4056x over previous
import jax
import jax.numpy as jnp
from jax import lax
from jax.experimental import pallas as pl
from jax.experimental.pallas import tpu as pltpu
from jax.experimental.pallas import tpu_sc as plsc

F32 = jnp.float32

NUM_NODES = 500
BATCH = 100
N = NUM_NODES * BATCH
E = 600000
NODE_DIM = 64
EMB = 128
EDGE_DIM = 16
NLAYER = 3

QW = 32
NQ = EMB // QW
EB = 128
NTILE = 32
EPAD = ((E + EB * NTILE - 1) // (EB * NTILE)) * (EB * NTILE)
NB = EPAD // EB
BPT = NB // NTILE
ACCR = ((N + 128) // 128) * 128
RPT = ACCR // 16
ZR = 136
ZFULL = RPT // ZR
ZREM = RPT - ZFULL * ZR


def _leaky(v):
    return jnp.where(v >= 0, v, 0.1 * v)


def _encoder_body(x_ref, w_ref, cb_ref, lw_ref, lb_ref,
                  h_ref, q0_ref, q1_ref, q2_ref, q3_ref):
    xb = x_ref[...]
    nn = xb.shape[0] // 2

    def shifts(m):
        z = jnp.zeros((nn, 1), F32)
        left = jnp.concatenate([z, m[:, :-1]], axis=1)
        right = jnp.concatenate([m[:, 1:], z], axis=1)
        return jnp.concatenate([left, m, right], axis=0)

    xcat = jnp.concatenate([shifts(xb[:nn]), shifts(xb[nn:])], axis=1)
    o = jnp.dot(w_ref[...], xcat, preferred_element_type=F32)
    o = _leaky(o + cb_ref[...])
    lw = lw_ref[...]
    lb = lb_ref[...]
    h0 = _leaky(jnp.dot(o[:, :NODE_DIM], lw, preferred_element_type=F32) + lb)
    h1 = _leaky(jnp.dot(o[:, NODE_DIM:], lw, preferred_element_type=F32) + lb)
    h = jnp.concatenate([h0, h1], axis=0)
    h_ref[...] = h
    q0_ref[...] = h[:, 0 * QW:1 * QW]
    q1_ref[...] = h[:, 1 * QW:2 * QW]
    q2_ref[...] = h[:, 2 * QW:3 * QW]
    q3_ref[...] = h[:, 3 * QW:4 * QW]


def _encoder(x, wcat, cb, lw, lb):
    nrow = 2 * NUM_NODES
    grid = (BATCH // 2,)
    qspec = pl.BlockSpec((nrow, QW), lambda i: (i, 0))
    return pl.pallas_call(
        _encoder_body,
        grid=grid,
        in_specs=[
            pl.BlockSpec((nrow, NODE_DIM), lambda i: (i, 0)),
            pl.BlockSpec((NUM_NODES, 3 * NUM_NODES), lambda i: (0, 0)),
            pl.BlockSpec((NUM_NODES, 1), lambda i: (0, 0)),
            pl.BlockSpec((NODE_DIM, EMB), lambda i: (0, 0)),
            pl.BlockSpec((1, EMB), lambda i: (0, 0)),
        ],
        out_specs=[pl.BlockSpec((nrow, EMB), lambda i: (i, 0)),
                   qspec, qspec, qspec, qspec],
        out_shape=[jax.ShapeDtypeStruct((N, EMB), F32)] +
                  [jax.ShapeDtypeStruct((N, QW), F32)] * NQ,
    )(x, wcat, cb, lw, lb)


def _ee_body(a4_ref, w0_ref, w1_ref, w2_ref, w3_ref,
             b0_ref, b1_ref, b2_ref, b3_ref,
             q0_ref, q1_ref, q2_ref, q3_ref):
    a4 = a4_ref[...]
    for wr, br, qr in ((w0_ref, b0_ref, q0_ref), (w1_ref, b1_ref, q1_ref),
                       (w2_ref, b2_ref, q2_ref), (w3_ref, b3_ref, q3_ref)):
        v = jnp.dot(a4, wr[...], preferred_element_type=F32)
        qr[...] = jnp.maximum(v + br[...], 0.0)


def _edge_embed(a4, w4s, b4s):
    blk = 1536
    grid = (EPAD // 4 // blk,)
    qspec = pl.BlockSpec((blk, EMB), lambda i: (i, 0))
    wspec = pl.BlockSpec((4 * EDGE_DIM, EMB), lambda i: (0, 0))
    bspec = pl.BlockSpec((1, EMB), lambda i: (0, 0))
    return pl.pallas_call(
        _ee_body,
        grid=grid,
        in_specs=[pl.BlockSpec((blk, 4 * EDGE_DIM), lambda i: (i, 0)),
                  wspec, wspec, wspec, wspec,
                  bspec, bspec, bspec, bspec],
        out_specs=[qspec] * NQ,
        out_shape=[jax.ShapeDtypeStruct((EPAD // 4, EMB), F32)] * NQ,
    )(a4, *w4s, *b4s)


def _sc_body(idx_ref,
             h0_ref, h1_ref, h2_ref, h3_ref,
             e0_ref, e1_ref, e2_ref, e3_ref,
             apk_ref,
             iv, hbuf, ebuf, mbuf, zbuf, acc, semi, semg, seme, sems):
    c = lax.axis_index("c")
    s = lax.axis_index("s")
    wid = s * 2 + c
    b0 = wid * BPT

    def zfill(i, _):
        zbuf[i, pl.ds(0, 16)] = jnp.zeros((16,), F32)
        zbuf[i, pl.ds(16, 16)] = jnp.zeros((16,), F32)
        return 0
    lax.fori_loop(0, ZR, zfill, 0)

    hq = (h0_ref, h1_ref, h2_ref, h3_ref)
    eq = (e0_ref, e1_ref, e2_ref, e3_ref)

    for q in range(NQ):
        base = s * RPT
        for j in range(ZFULL):
            pltpu.async_copy(zbuf, acc.at[pl.ds(base + j * ZR, ZR)],
                             semg.at[0])
        for j in range(ZFULL):
            pltpu.make_async_copy(zbuf, acc.at[pl.ds(base + j * ZR, ZR)],
                                  semg.at[0]).wait()
        if ZREM:
            pltpu.sync_copy(zbuf.at[pl.ds(0, ZREM)],
                            acc.at[pl.ds(base + ZFULL * ZR, ZREM)])
        plsc.subcore_barrier()

        def idx_desc(j):
            sl = lax.rem(j, 4)
            return pltpu.make_async_copy(
                idx_ref.at[pl.ds(b0 + j, 1)], iv.at[pl.ds(sl, 1)],
                semi.at[sl])

        def gather_desc(j):
            return pltpu.make_async_copy(
                hq[q].at[iv.at[lax.rem(j, 4), 0]],
                hbuf.at[lax.rem(j, 2)], semg.at[lax.rem(j, 2)])

        def ee_desc(j):
            return pltpu.make_async_copy(
                eq[q].at[pl.ds((b0 + j) * QW, QW)],
                ebuf.at[lax.rem(j, 2)], seme.at[lax.rem(j, 2)])

        def scat_desc(j):
            return pltpu.make_async_copy(
                mbuf.at[lax.rem(j, 2)], acc.at[iv.at[lax.rem(j, 4), 1]],
                sems.at[lax.rem(j, 2)])

        idx_desc(0).start()
        idx_desc(1).start()
        idx_desc(0).wait()
        gather_desc(0).start()
        ee_desc(0).start()

        def batch_body(i, _):
            sl = lax.rem(i, 2)

            @pl.when(i >= 2)
            def _():
                scat_desc(i - 2).wait()

            @pl.when(i + 2 < BPT)
            def _():
                idx_desc(i + 2).start()

            @pl.when(i + 1 < BPT)
            def _():
                idx_desc(i + 1).wait()
                gather_desc(i + 1).start()
                ee_desc(i + 1).start()

            gather_desc(i).wait()
            ee_desc(i).wait()

            for slv in (0, 1):
                @pl.when(sl == slv)
                def _(slv=slv):
                    def ebody(t, _):
                        for dr in range(4):
                            er = t * 4 + dr
                            for j in range(4):
                                for kk in (0, 16):
                                    v = (hbuf[slv, er * 4 + j,
                                              pl.ds(kk, 16)] +
                                         ebuf[slv, er,
                                              pl.ds(j * QW + kk, 16)])
                                    mbuf[slv, er * 4 + j, pl.ds(kk, 16)] = (
                                        jnp.maximum(v, 0.0))
                        return 0
                    lax.fori_loop(0, QW // 4, ebody, 0)

            pltpu.async_copy(mbuf.at[sl], acc.at[iv.at[lax.rem(i, 4), 1]],
                             sems.at[lax.rem(i, 2)], add=True)
            return 0
        lax.fori_loop(0, BPT, batch_body, 0)

        scat_desc(BPT - 2).wait()
        scat_desc(BPT - 1).wait()

        plsc.subcore_barrier()
        pltpu.sync_copy(acc.at[pl.ds(s * RPT, RPT)],
                        apk_ref.at[c, pl.ds(s * RPT, RPT),
                                   pl.ds(q * QW, QW)])
        plsc.subcore_barrier()


def _sc_messages(idx2, hqs, eqs):
    mesh = plsc.VectorSubcoreMesh(core_axis_name="c", subcore_axis_name="s")
    fn = pl.kernel(
        _sc_body,
        out_type=jax.ShapeDtypeStruct((2, ACCR, EMB), F32),
        mesh=mesh,
        compiler_params=pltpu.CompilerParams(use_tc_tiling_on_sc=False),
        scratch_types=[
            pltpu.VMEM((4, 2, EB), jnp.int32),
            pltpu.VMEM((2, EB, QW), F32),
            pltpu.VMEM((2, QW, EMB), F32),
            pltpu.VMEM((2, EB, QW), F32),
            pltpu.VMEM((ZR, QW), F32),
            pltpu.VMEM_SHARED((ACCR, QW), F32),
            pltpu.SemaphoreType.DMA((4,)),
            pltpu.SemaphoreType.DMA((2,)),
            pltpu.SemaphoreType.DMA((2,)),
            pltpu.SemaphoreType.DMA((2,)),
        ],
    )
    return fn(idx2, *hqs, *eqs)


def _mlpa_body(h_ref, a_ref, k_ref,
               w1_ref, b1_ref, z_ref, s_ref, s2_ref):
    agg = a_ref[0] + a_ref[1]
    t = h_ref[...] * k_ref[...] + agg
    z = jnp.dot(t, w1_ref[...], preferred_element_type=F32) + b1_ref[...]
    z_ref[...] = z
    s = jnp.sum(z, axis=0, keepdims=True)
    s2 = jnp.sum(z * z, axis=0, keepdims=True)

    @pl.when(pl.program_id(0) == 0)
    def _():
        s_ref[...] = s
        s2_ref[...] = s2

    @pl.when(pl.program_id(0) != 0)
    def _():
        s_ref[...] += s
        s2_ref[...] += s2


def _mlp_a(h, agg, kmul, w1, b1):
    rblk = 2000
    grid = (N // rblk,)
    return pl.pallas_call(
        _mlpa_body,
        grid=grid,
        in_specs=[
            pl.BlockSpec((rblk, EMB), lambda i: (i, 0)),
            pl.BlockSpec((2, rblk, EMB), lambda i: (0, i, 0)),
            pl.BlockSpec((1, 1), lambda i: (0, 0)),
            pl.BlockSpec((EMB, 2 * EMB), lambda i: (0, 0)),
            pl.BlockSpec((1, 2 * EMB), lambda i: (0, 0)),
        ],
        out_specs=[
            pl.BlockSpec((rblk, 2 * EMB), lambda i: (i, 0)),
            pl.BlockSpec((1, 2 * EMB), lambda i: (0, 0)),
            pl.BlockSpec((1, 2 * EMB), lambda i: (0, 0)),
        ],
        out_shape=[
            jax.ShapeDtypeStruct((N, 2 * EMB), F32),
            jax.ShapeDtypeStruct((1, 2 * EMB), F32),
            jax.ShapeDtypeStruct((1, 2 * EMB), F32),
        ],
    )(h, agg, kmul, w1, b1)


def _mlpb_body(z_ref, h_ref, jk_ref, s_ref, s2_ref, bg_ref, bb_ref,
               w2_ref, b2_ref, lg_ref, lb_ref,
               hn_ref, jko_ref, q0_ref, q1_ref, q2_ref, q3_ref):
    inv_n = 1.0 / N
    mu = s_ref[...] * inv_n
    var = s2_ref[...] * inv_n - mu * mu
    scale = bg_ref[...] * lax.rsqrt(var + 1e-5)
    zb = (z_ref[...] - mu) * scale + bb_ref[...]
    zb = jnp.maximum(zb, 0.0)
    y = jnp.dot(zb, w2_ref[...], preferred_element_type=F32) + b2_ref[...]
    m = jnp.mean(y, axis=1, keepdims=True)
    v = jnp.mean(y * y, axis=1, keepdims=True) - m * m
    y = (y - m) * lax.rsqrt(v + 1e-5) * lg_ref[...] + lb_ref[...]
    y = _leaky(y)
    hn = y + h_ref[...]
    hn_ref[...] = hn
    jko_ref[...] = jk_ref[...] + hn
    q0_ref[...] = hn[:, 0 * QW:1 * QW]
    q1_ref[...] = hn[:, 1 * QW:2 * QW]
    q2_ref[...] = hn[:, 2 * QW:3 * QW]
    q3_ref[...] = hn[:, 3 * QW:4 * QW]


def _mlp_b(z, h, jk, s, s2, bg, bb, w2, b2, lg, lb):
    rblk = 2000
    grid = (N // rblk,)
    qspec = pl.BlockSpec((rblk, QW), lambda i: (i, 0))
    row128 = pl.BlockSpec((1, EMB), lambda i: (0, 0))
    row256 = pl.BlockSpec((1, 2 * EMB), lambda i: (0, 0))
    return pl.pallas_call(
        _mlpb_body,
        grid=grid,
        in_specs=[
            pl.BlockSpec((rblk, 2 * EMB), lambda i: (i, 0)),
            pl.BlockSpec((rblk, EMB), lambda i: (i, 0)),
            pl.BlockSpec((rblk, EMB), lambda i: (i, 0)),
            row256, row256, row256, row256,
            pl.BlockSpec((2 * EMB, EMB), lambda i: (0, 0)),
            row128, row128, row128,
        ],
        out_specs=[pl.BlockSpec((rblk, EMB), lambda i: (i, 0)),
                   pl.BlockSpec((rblk, EMB), lambda i: (i, 0)),
                   qspec, qspec, qspec, qspec],
        out_shape=[jax.ShapeDtypeStruct((N, EMB), F32),
                   jax.ShapeDtypeStruct((N, EMB), F32)] +
                  [jax.ShapeDtypeStruct((N, QW), F32)] * NQ,
    )(z, h, jk, s, s2, bg, bb, w2, b2, lg, lb)


@jax.jit
def kernel(x, edge_index, edge_attr, conv_w, conv_b, enc_lin_w, enc_lin_b,
           mlp_w1, mlp_b1, bn_gamma, bn_beta, mlp_w2, mlp_b2, eps,
           edge_w, edge_b, ln_gamma, ln_beta):
    wcat = jnp.concatenate([conv_w[:, :, k] for k in range(3)], axis=1)
    cb = conv_b[:, None]
    lb = enc_lin_b[None, :]

    row = edge_index[0]
    col = edge_index[1]
    pad = EPAD - E
    rows2d = jnp.concatenate(
        [row, jnp.zeros((pad,), jnp.int32)]).reshape(NB, EB)
    cols2d = jnp.concatenate(
        [col, jnp.full((pad,), N, jnp.int32)]).reshape(NB, EB)
    idx2 = jnp.stack([rows2d, cols2d], axis=1)
    a4 = jnp.concatenate(
        [edge_attr, jnp.zeros((pad, EDGE_DIM), F32)],
        axis=0).reshape(EPAD // 4, 4 * EDGE_DIM)
    eye4 = jnp.eye(4, dtype=F32)

    eqs_all = []
    for l in range(NLAYER):
        w4s = [jnp.kron(eye4, edge_w[l][:, q * QW:(q + 1) * QW])
               for q in range(NQ)]
        b4s = [jnp.tile(edge_b[l][q * QW:(q + 1) * QW], 4)[None, :]
               for q in range(NQ)]
        eqs_all.append(_edge_embed(a4, w4s, b4s))

    h, *hqs = _encoder(x, wcat, cb, enc_lin_w, lb)
    jk = h
    for l in range(NLAYER):
        agg = _sc_messages(idx2, hqs, eqs_all[l])
        kmul = (1.0 + eps[l]).reshape(1, 1)
        z, s, s2 = _mlp_a(h, agg, kmul, mlp_w1[l], mlp_b1[l][None, :])
        h, jk, *hqs = _mlp_b(z, h, jk, s, s2,
                             bn_gamma[l][None, :], bn_beta[l][None, :],
                             mlp_w2[l], mlp_b2[l][None, :],
                             ln_gamma[l][None, :], ln_beta[l][None, :])
    return jk
